# Initial kernel scaffold; baseline (speedup 1.0000x reference)
import jax
import jax.numpy as jnp
from jax.experimental import pallas as pl


def kernel(x, Wg, bg, W1, b1, W2, b2, gamma, beta):
    raise NotImplementedError("write your pallas kernel here")


# speedup vs baseline: 1.2558x; 1.2558x over previous
import functools

import jax
import jax.numpy as jnp
from jax.experimental import pallas as pl
from jax.experimental.pallas import tpu as pltpu

D = 768
E = 8
DFF = 2 * D
N = 2048
TN = 256


def _moe_body(x_ref, wg_ref, bg_ref, w1_ref, b1_ref, w2_ref, b2_ref,
              gamma_ref, beta_ref, out_ref, gw_ref):
    e = pl.program_id(1)
    xt = x_ref[...]

    @pl.when(e == 0)
    def _gating():
        logits = jax.lax.dot_general(
            xt, wg_ref[...],
            dimension_numbers=(((1,), (1,)), ((), ())),
            preferred_element_type=jnp.float32,
        ) + bg_ref[...]
        ii = jax.lax.broadcasted_iota(jnp.int32, (TN, E), 1)
        v1 = jnp.max(logits, axis=1, keepdims=True)
        i1 = jnp.min(jnp.where(logits == v1, ii, E), axis=1, keepdims=True)
        m1 = ii == i1
        neg = jnp.where(m1, -jnp.inf, logits)
        v2 = jnp.max(neg, axis=1, keepdims=True)
        i2 = jnp.min(jnp.where(neg == v2, ii, E), axis=1, keepdims=True)
        m2 = ii == i2
        z = jnp.exp(v2 - v1)
        sm1 = 1.0 / (1.0 + z)
        sm2 = z / (1.0 + z)
        gw_ref[...] = jnp.where(m1, sm1, 0.0) + jnp.where(m2, sm2, 0.0)

    ii = jax.lax.broadcasted_iota(jnp.int32, (TN, E), 1)
    wcol = jnp.sum(jnp.where(ii == e, gw_ref[...], 0.0), axis=1,
                   keepdims=True)

    xb = xt.astype(jnp.bfloat16)
    w1 = w1_ref[e]
    h = jax.lax.dot_general(
        xb, w1, dimension_numbers=(((1,), (1,)), ((), ())),
        preferred_element_type=jnp.float32,
    ) + b1_ref[pl.ds(e, 1), :]
    h = jnp.maximum(h, 0.0)
    hb = h.astype(jnp.bfloat16)
    w2 = w2_ref[e]
    y = jax.lax.dot_general(
        hb, w2, dimension_numbers=(((1,), (1,)), ((), ())),
        preferred_element_type=jnp.float32,
    ) + b2_ref[pl.ds(e, 1), :]
    contrib = y * wcol

    @pl.when(e == 0)
    def _init():
        out_ref[...] = xt + contrib

    @pl.when(jnp.logical_and(e > 0, e < E - 1))
    def _acc():
        out_ref[...] = out_ref[...] + contrib

    @pl.when(e == E - 1)
    def _final():
        res = out_ref[...] + contrib
        mu = jnp.mean(res, axis=1, keepdims=True)
        var = jnp.mean((res - mu) ** 2, axis=1, keepdims=True)
        out_ref[...] = (gamma_ref[...] * (res - mu)
                        * jax.lax.rsqrt(var + 1e-5) + beta_ref[...])


@jax.jit
def kernel(x, Wg, bg, W1, b1, W2, b2, gamma, beta):
    w1b = W1.astype(jnp.bfloat16)
    w2b = W2.astype(jnp.bfloat16)
    bg2 = bg.reshape(1, E)
    gamma2 = gamma.reshape(1, D)
    beta2 = beta.reshape(1, D)
    nt = N // TN
    out = pl.pallas_call(
        _moe_body,
        grid=(nt, E),
        in_specs=[
            pl.BlockSpec((TN, D), lambda t, e: (t, 0)),
            pl.BlockSpec((E, D), lambda t, e: (0, 0)),
            pl.BlockSpec((1, E), lambda t, e: (0, 0)),
            pl.BlockSpec((E, DFF, D), lambda t, e: (0, 0, 0)),
            pl.BlockSpec((E, DFF), lambda t, e: (0, 0)),
            pl.BlockSpec((E, D, DFF), lambda t, e: (0, 0, 0)),
            pl.BlockSpec((E, D), lambda t, e: (0, 0)),
            pl.BlockSpec((1, D), lambda t, e: (0, 0)),
            pl.BlockSpec((1, D), lambda t, e: (0, 0)),
        ],
        out_specs=pl.BlockSpec((TN, D), lambda t, e: (t, 0)),
        out_shape=jax.ShapeDtypeStruct((N, D), jnp.float32),
        scratch_shapes=[pltpu.VMEM((TN, E), jnp.float32)],
        compiler_params=pltpu.CompilerParams(
            dimension_semantics=("parallel", "arbitrary"),
        ),
    )(x, Wg, bg2, w1b, b1, w2b, b2, gamma2, beta2)
    return out
